# baseline (speedup 1.0000x reference)
import functools

import jax
import jax.numpy as jnp
from jax import lax
from jax.experimental import pallas as pl
from jax.experimental.pallas import tpu as pltpu, tpu_sc as plsc

_NC = 2
_NS = 16
_LANES = 16
_NW = _NC * _NS
_BATCH = 16384
_DIM = 64
_BPW = _BATCH // _NW
_NIDX = 4
_CHUNK = _BPW // _NIDX
_GROUPS = _CHUNK // _LANES


def _body(u_hbm, i_hbm, uft_hbm, ift_hbm, ub_hbm, ib_hbm, gb_hbm, out_hbm,
          idx_u, idx_i, ifu0, ifu1, ifi0, ifi1,
          ubuf0, ubuf1, ibuf0, ibuf1, acc, ub_v, ib_v, gb_v,
          sem0, sem1, semb):
    wid = lax.axis_index("s") * _NC + lax.axis_index("c")
    base = wid * _BPW

    for j in range(_NIDX):
        pltpu.sync_copy(u_hbm.at[pl.ds(base + j * _CHUNK, _CHUNK)], idx_u.at[j])
        pltpu.sync_copy(i_hbm.at[pl.ds(base + j * _CHUNK, _CHUNK)], idx_i.at[j])
    pltpu.sync_copy(gb_hbm, gb_v)

    bias_copies = []
    for j in range(_NIDX):
        bias_copies.append(
            pltpu.async_copy(ub_hbm.at[idx_u.at[j]], ub_v.at[j], semb))
        bias_copies.append(
            pltpu.async_copy(ib_hbm.at[idx_i.at[j]], ib_v.at[j], semb))

    gb = gb_v[...]
    for j in range(_NIDX):
        for v in range(_GROUPS):
            acc[j, pl.ds(v * _LANES, _LANES)] = gb

    for c in bias_copies:
        c.wait()

    def fire(d, ifu, ifi, ub, ib, sem):
        off = d * jnp.int32(1000000)
        for j in range(_NIDX):
            for v in range(_GROUPS):
                sl = pl.ds(v * _LANES, _LANES)
                ifu[j, sl] = idx_u[j, sl] + off
                ifi[j, sl] = idx_i[j, sl] + off
        hs = []
        for j in range(_NIDX):
            hs.append(pltpu.async_copy(
                uft_hbm.at[ifu.at[j]], ub.at[j], sem))
            hs.append(pltpu.async_copy(
                ift_hbm.at[ifi.at[j]], ib.at[j], sem))
        return hs

    def accum(ub, ib):
        for j in range(_NIDX):
            for v in range(_GROUPS):
                sl = pl.ds(v * _LANES, _LANES)
                acc[j, sl] = acc[j, sl] + ub[j, sl] * ib[j, sl]

    def body(t, carry):
        d = 2 * t
        h0 = fire(d, ifu0, ifi0, ubuf0, ibuf0, sem0)
        h1 = fire(d + 1, ifu1, ifi1, ubuf1, ibuf1, sem1)
        for h in h0:
            h.wait()
        accum(ubuf0, ibuf0)
        for h in h1:
            h.wait()
        accum(ubuf1, ibuf1)
        return carry

    lax.fori_loop(0, _DIM // 2, body, 0)

    for j in range(_NIDX):
        for v in range(_GROUPS):
            sl = pl.ds(v * _LANES, _LANES)
            acc[j, sl] = acc[j, sl] + ub_v[j, sl] + ib_v[j, sl]
    for j in range(_NIDX):
        pltpu.sync_copy(acc.at[j], out_hbm.at[pl.ds(base + j * _CHUNK, _CHUNK)])


_mbcf = functools.partial(
    pl.kernel,
    out_type=jax.ShapeDtypeStruct((_BATCH,), jnp.float32),
    mesh=plsc.VectorSubcoreMesh(core_axis_name="c", subcore_axis_name="s"),
    compiler_params=pltpu.CompilerParams(needs_layout_passes=False,
                                         use_tc_tiling_on_sc=True),
    scratch_types=[
        pltpu.VMEM((_NIDX, _CHUNK), jnp.int32),
        pltpu.VMEM((_NIDX, _CHUNK), jnp.int32),
        pltpu.VMEM((_NIDX, _CHUNK), jnp.int32),
        pltpu.VMEM((_NIDX, _CHUNK), jnp.int32),
        pltpu.VMEM((_NIDX, _CHUNK), jnp.int32),
        pltpu.VMEM((_NIDX, _CHUNK), jnp.int32),
        pltpu.VMEM((_NIDX, _CHUNK), jnp.float32),
        pltpu.VMEM((_NIDX, _CHUNK), jnp.float32),
        pltpu.VMEM((_NIDX, _CHUNK), jnp.float32),
        pltpu.VMEM((_NIDX, _CHUNK), jnp.float32),
        pltpu.VMEM((_NIDX, _CHUNK), jnp.float32),
        pltpu.VMEM((_NIDX, _CHUNK), jnp.float32),
        pltpu.VMEM((_NIDX, _CHUNK), jnp.float32),
        pltpu.VMEM((_LANES,), jnp.float32),
        pltpu.SemaphoreType.DMA,
        pltpu.SemaphoreType.DMA,
        pltpu.SemaphoreType.DMA,
    ],
)(_body)


@jax.jit
def kernel(u, i, user_factors, item_factors, user_bias, item_bias, global_bias):
    gb16 = jnp.broadcast_to(global_bias.astype(jnp.float32), (_LANES,))
    uft = user_factors.T.reshape(-1)
    ift = item_factors.T.reshape(-1)
    return _mbcf(u.astype(jnp.int32), i.astype(jnp.int32), uft, ift,
                 user_bias.reshape(-1), item_bias.reshape(-1), gb16)


# speedup vs baseline: 8.9449x; 8.9449x over previous
import functools

import jax
import jax.numpy as jnp
from jax import lax
from jax.experimental import pallas as pl
from jax.experimental.pallas import tpu as pltpu, tpu_sc as plsc

_NC = 2
_NS = 16
_LANES = 16
_NW = _NC * _NS
_BATCH = 16384
_DIM = 64
_BPW = _BATCH // _NW
_CHUNK = 128
_NCHUNK = _BPW // _CHUNK
_GROUPS = _CHUNK // _LANES


def _body(u_hbm, i_hbm, uf_hbm, if_hbm, ub_hbm, ib_hbm, gb_hbm, out_hbm,
          idx_u, idx_i, row_u, row_i, uf0, uf1, if0, if1,
          ub_v, ib_v, gb_v, out_v, sem0, sem1, semb):
    wid = lax.axis_index("s") * _NC + lax.axis_index("c")
    base = wid * _BPW

    for j in range(_NCHUNK):
        pltpu.sync_copy(u_hbm.at[pl.ds(base + j * _CHUNK, _CHUNK)], idx_u.at[j])
        pltpu.sync_copy(i_hbm.at[pl.ds(base + j * _CHUNK, _CHUNK)], idx_i.at[j])
    pltpu.sync_copy(gb_hbm, gb_v)

    for j in range(_NCHUNK):
        for v in range(_GROUPS):
            sl = pl.ds(v * _LANES, _LANES)
            row_u[j, sl] = idx_u[j, sl] >> 1
            row_i[j, sl] = idx_i[j, sl] >> 1

    bias_copies = []
    for j in range(_NCHUNK):
        sl = pl.ds(j * _CHUNK, _CHUNK)
        bias_copies.append(pltpu.async_copy(ub_hbm.at[idx_u.at[j]], ub_v.at[sl], semb))
        bias_copies.append(pltpu.async_copy(ib_hbm.at[idx_i.at[j]], ib_v.at[sl], semb))

    ubufs = (uf0, uf1)
    ibufs = (if0, if1)
    sems = (sem0, sem1)

    def fire(p):
        s = sems[p % 2]
        return (pltpu.async_copy(uf_hbm.at[row_u.at[p]], ubufs[p % 2], s),
                pltpu.async_copy(if_hbm.at[row_i.at[p]], ibufs[p % 2], s))

    inflight = fire(0)
    for c in bias_copies:
        c.wait()

    lanes = lax.iota(jnp.int32, _LANES)
    gb = gb_v[...]
    one = jnp.full((_LANES,), 1, jnp.int32)

    for p in range(_NCHUNK):
        for c in inflight:
            c.wait()
        if p + 1 < _NCHUNK:
            inflight = fire(p + 1)
        ubuf = ubufs[p % 2]
        ibuf = ibufs[p % 2]

        def group(g, carry, p=p, ubuf=ubuf, ibuf=ibuf):
            rows = g * _LANES + lanes
            gsl = pl.ds(g * _LANES, _LANES)
            colu = (idx_u[p, gsl] & one) << 6
            coli = (idx_i[p, gsl] & one) << 6
            acc = gb
            for d in range(_DIM):
                acc = acc + (plsc.load_gather(ubuf, [rows, colu + d])
                             * plsc.load_gather(ibuf, [rows, coli + d]))
            out_v[pl.ds(p * _CHUNK + g * _LANES, _LANES)] = acc
            return carry

        lax.fori_loop(0, _GROUPS, group, 0)

    for v in range(_BPW // _LANES):
        sl = pl.ds(v * _LANES, _LANES)
        out_v[sl] = out_v[sl] + ub_v[sl] + ib_v[sl]

    pltpu.sync_copy(out_v, out_hbm.at[pl.ds(base, _BPW)])


_mbcf = functools.partial(
    pl.kernel,
    out_type=jax.ShapeDtypeStruct((_BATCH,), jnp.float32),
    mesh=plsc.VectorSubcoreMesh(core_axis_name="c", subcore_axis_name="s"),
    compiler_params=pltpu.CompilerParams(needs_layout_passes=False,
                                         use_tc_tiling_on_sc=True),
    scratch_types=[
        pltpu.VMEM((_NCHUNK, _CHUNK), jnp.int32),
        pltpu.VMEM((_NCHUNK, _CHUNK), jnp.int32),
        pltpu.VMEM((_NCHUNK, _CHUNK), jnp.int32),
        pltpu.VMEM((_NCHUNK, _CHUNK), jnp.int32),
        pltpu.VMEM((_CHUNK, 2 * _DIM), jnp.float32),
        pltpu.VMEM((_CHUNK, 2 * _DIM), jnp.float32),
        pltpu.VMEM((_CHUNK, 2 * _DIM), jnp.float32),
        pltpu.VMEM((_CHUNK, 2 * _DIM), jnp.float32),
        pltpu.VMEM((_BPW,), jnp.float32),
        pltpu.VMEM((_BPW,), jnp.float32),
        pltpu.VMEM((_LANES,), jnp.float32),
        pltpu.VMEM((_BPW,), jnp.float32),
        pltpu.SemaphoreType.DMA,
        pltpu.SemaphoreType.DMA,
        pltpu.SemaphoreType.DMA,
    ],
)(_body)


@jax.jit
def kernel(u, i, user_factors, item_factors, user_bias, item_bias, global_bias):
    gb16 = jnp.broadcast_to(global_bias.astype(jnp.float32), (_LANES,))
    ufr = user_factors.reshape(-1, 2 * _DIM)
    ifr = item_factors.reshape(-1, 2 * _DIM)
    return _mbcf(u.astype(jnp.int32), i.astype(jnp.int32), ufr, ifr,
                 user_bias.reshape(-1), item_bias.reshape(-1), gb16)
